# Initial kernel scaffold; baseline (speedup 1.0000x reference)
import jax
import jax.numpy as jnp
from jax.experimental import pallas as pl


def kernel(x, edge_index, W1, b1, W2, b2):
    raise NotImplementedError("write your pallas kernel here")


# speedup vs baseline: 19.7051x; 19.7051x over previous
import functools

import jax
import jax.numpy as jnp
from jax import lax
from jax.experimental import pallas as pl
from jax.experimental.pallas import tpu as pltpu
from jax.experimental.pallas import tpu_sc as plsc

N_NODES = 10000
N_PAD = 10240
N_EDGES = 320000
D_IN = 128
D_HID = 64
D_OUT = 32

NC = 2
NS = 16
NW = NC * NS
EPW = N_EDGES // NW
CHUNK = 128
FULL_CHUNKS = EPW // CHUNK
REM = EPW - FULL_CHUNKS * CHUNK
ROWS_PER_TILE = N_PAD // NS


def _mesh():
    return plsc.VectorSubcoreMesh(core_axis_name="c", subcore_axis_name="s")


@functools.lru_cache(maxsize=None)
def _deg_kernel():
    @functools.partial(
        pl.kernel,
        mesh=_mesh(),
        out_type=jax.ShapeDtypeStruct((NC, N_PAD), jnp.float32),
        scratch_types=[
            pltpu.VMEM((CHUNK,), jnp.float32),
            pltpu.VMEM((CHUNK,), jnp.int32),
            pltpu.VMEM((REM,), jnp.float32),
            pltpu.VMEM((REM,), jnp.int32),
            pltpu.VMEM_SHARED((N_PAD,), jnp.float32),
        ],
    )
    def deg(dst_hbm, zeros_hbm, out_hbm, ones_v, idx_v, ones_r, idx_r, acc):
        cid = lax.axis_index("c")
        sid = lax.axis_index("s")
        wid = sid * NC + cid
        for j in range(CHUNK // 16):
            ones_v[pl.ds(j * 16, 16)] = jnp.full((16,), 1.0, jnp.float32)
        ones_r[...] = jnp.full((REM,), 1.0, jnp.float32)
        row0 = sid * ROWS_PER_TILE
        pltpu.sync_copy(zeros_hbm.at[pl.ds(row0, ROWS_PER_TILE)],
                        acc.at[pl.ds(row0, ROWS_PER_TILE)])
        plsc.subcore_barrier()
        ebase = wid * EPW

        def body(i, carry):
            base = pl.multiple_of(ebase + i * CHUNK, 8)
            pltpu.sync_copy(dst_hbm.at[pl.ds(base, CHUNK)], idx_v)
            pltpu.sync_copy(ones_v, acc.at[idx_v], add=True)
            return carry

        lax.fori_loop(0, FULL_CHUNKS, body, 0)
        rbase = pl.multiple_of(ebase + FULL_CHUNKS * CHUNK, 8)
        pltpu.sync_copy(dst_hbm.at[pl.ds(rbase, REM)], idx_r)
        pltpu.sync_copy(ones_r, acc.at[idx_r], add=True)
        plsc.subcore_barrier()
        pltpu.sync_copy(acc.at[pl.ds(row0, ROWS_PER_TILE)],
                        out_hbm.at[cid, pl.ds(row0, ROWS_PER_TILE)])

    return deg


@functools.lru_cache(maxsize=None)
def _prop_kernel(d: int):
    @functools.partial(
        pl.kernel,
        mesh=_mesh(),
        out_type=jax.ShapeDtypeStruct((NC, N_PAD, d), jnp.float32),
        scratch_types=[
            pltpu.VMEM((CHUNK,), jnp.int32),
            pltpu.VMEM((CHUNK,), jnp.int32),
            pltpu.VMEM((CHUNK, d), jnp.float32),
            pltpu.VMEM((REM,), jnp.int32),
            pltpu.VMEM((REM,), jnp.int32),
            pltpu.VMEM((REM, d), jnp.float32),
            pltpu.VMEM_SHARED((N_PAD, d), jnp.float32),
            pltpu.SemaphoreType.DMA,
        ],
        compiler_params=pltpu.CompilerParams(use_tc_tiling_on_sc=False),
    )
    def prop(hs_hbm, src_hbm, dst_hbm, zeros_hbm, out_hbm,
             sidx, didx, rows, sidx_r, didx_r, rows_r, acc, sem):
        cid = lax.axis_index("c")
        sid = lax.axis_index("s")
        wid = sid * NC + cid
        row0 = sid * ROWS_PER_TILE
        pltpu.sync_copy(zeros_hbm.at[pl.ds(row0, ROWS_PER_TILE)],
                        acc.at[pl.ds(row0, ROWS_PER_TILE)])
        plsc.subcore_barrier()
        ebase = wid * EPW

        def body(i, carry):
            base = pl.multiple_of(ebase + i * CHUNK, 8)
            pltpu.sync_copy(src_hbm.at[pl.ds(base, CHUNK)], sidx)
            pltpu.sync_copy(dst_hbm.at[pl.ds(base, CHUNK)], didx)
            pltpu.async_copy(hs_hbm.at[sidx], rows, sem).wait()
            pltpu.sync_copy(rows, acc.at[didx], add=True)
            return carry

        lax.fori_loop(0, FULL_CHUNKS, body, 0)
        rbase = pl.multiple_of(ebase + FULL_CHUNKS * CHUNK, 8)
        pltpu.sync_copy(src_hbm.at[pl.ds(rbase, REM)], sidx_r)
        pltpu.sync_copy(dst_hbm.at[pl.ds(rbase, REM)], didx_r)
        pltpu.async_copy(hs_hbm.at[sidx_r], rows_r, sem).wait()
        pltpu.sync_copy(rows_r, acc.at[didx_r], add=True)
        plsc.subcore_barrier()
        pltpu.sync_copy(acc.at[pl.ds(row0, ROWS_PER_TILE)],
                        out_hbm.at[cid, pl.ds(row0, ROWS_PER_TILE)])

    return prop


def _mm_scale(x_ref, w_ref, h0_ref, h1_ref, hs_ref, dis_ref):
    dis = lax.rsqrt(h0_ref[...] + h1_ref[...] + 1.0)
    dis_ref[...] = dis
    hs_ref[...] = jnp.dot(x_ref[...], w_ref[...],
                          preferred_element_type=jnp.float32) * dis


def _layer2(p_ref, hs_ref, dis_ref, b1_ref, w2_ref, out_ref):
    agg = (p_ref[0] + p_ref[1] + hs_ref[...]) * dis_ref[...] + b1_ref[...]
    h = jnp.maximum(agg, 0.0)
    out_ref[...] = jnp.dot(h, w2_ref[...],
                           preferred_element_type=jnp.float32) * dis_ref[...]


def _final(q_ref, hs_ref, dis_ref, b2_ref, out_ref):
    out_ref[...] = ((q_ref[0] + q_ref[1] + hs_ref[...]) * dis_ref[...]
                    + b2_ref[...])


def kernel(x, edge_index, W1, b1, W2, b2):
    ei = edge_index.astype(jnp.int32)
    src = ei[0]
    dst = ei[1]
    x_p = jnp.pad(x, ((0, N_PAD - N_NODES), (0, 0)))
    z1 = jnp.zeros((N_PAD,), jnp.float32)
    zh = jnp.zeros((N_PAD, D_HID), jnp.float32)
    zo = jnp.zeros((N_PAD, D_OUT), jnp.float32)

    hist = _deg_kernel()(dst, z1)
    h0 = hist[0].reshape(N_PAD, 1)
    h1 = hist[1].reshape(N_PAD, 1)

    hs1, dis = pl.pallas_call(
        _mm_scale,
        out_shape=[jax.ShapeDtypeStruct((N_PAD, D_HID), jnp.float32),
                   jax.ShapeDtypeStruct((N_PAD, 1), jnp.float32)],
    )(x_p, W1, h0, h1)

    p = _prop_kernel(D_HID)(hs1, src, dst, zh)

    hs2 = pl.pallas_call(
        _layer2,
        out_shape=jax.ShapeDtypeStruct((N_PAD, D_OUT), jnp.float32),
    )(p, hs1, dis, b1.reshape(1, D_HID), W2)

    q = _prop_kernel(D_OUT)(hs2, src, dst, zo)

    out = pl.pallas_call(
        _final,
        out_shape=jax.ShapeDtypeStruct((N_PAD, D_OUT), jnp.float32),
    )(q, hs2, dis, b2.reshape(1, D_OUT))
    return out[:N_NODES]


# speedup vs baseline: 44.8551x; 2.2763x over previous
import functools

import jax
import jax.numpy as jnp
from jax import lax
from jax.experimental import pallas as pl
from jax.experimental.pallas import tpu as pltpu
from jax.experimental.pallas import tpu_sc as plsc

N_NODES = 10000
N_PAD = 10240
N_EDGES = 320000
D_IN = 128
D_HID = 64
D_OUT = 32

NC = 2
NS = 16
NW = NC * NS
EPW = N_EDGES // NW
CHUNK = 128
FULL_CHUNKS = EPW // CHUNK
REM = EPW - FULL_CHUNKS * CHUNK
ROWS_PER_TILE = N_PAD // NS
NBUF = 3


def _mesh():
    return plsc.VectorSubcoreMesh(core_axis_name="c", subcore_axis_name="s")


@functools.lru_cache(maxsize=None)
def _deg_kernel():
    @functools.partial(
        pl.kernel,
        mesh=_mesh(),
        out_type=jax.ShapeDtypeStruct((NC, N_PAD), jnp.float32),
        scratch_types=(
            [pltpu.VMEM((CHUNK,), jnp.float32)]
            + [pltpu.VMEM((CHUNK,), jnp.int32) for _ in range(NBUF)]
            + [pltpu.VMEM((REM,), jnp.float32),
               pltpu.VMEM((REM,), jnp.int32),
               pltpu.VMEM_SHARED((N_PAD,), jnp.float32)]
            + [pltpu.SemaphoreType.DMA for _ in range(NBUF)]
        ),
    )
    def deg(dst_hbm, zeros_hbm, out_hbm, ones_v, d0, d1, d2,
            ones_r, idx_r, acc, sd0, sd1, sd2):
        dbufs = (d0, d1, d2)
        dsems = (sd0, sd1, sd2)
        cid = lax.axis_index("c")
        sid = lax.axis_index("s")
        wid = sid * NC + cid
        for j in range(CHUNK // 16):
            ones_v[pl.ds(j * 16, 16)] = jnp.full((16,), 1.0, jnp.float32)
        ones_r[...] = jnp.full((REM,), 1.0, jnp.float32)
        row0 = sid * ROWS_PER_TILE
        pltpu.sync_copy(zeros_hbm.at[pl.ds(row0, ROWS_PER_TILE)],
                        acc.at[pl.ds(row0, ROWS_PER_TILE)])
        ebase = wid * EPW
        for b in range(NBUF):
            base = pl.multiple_of(ebase + b * CHUNK, 8)
            pltpu.async_copy(dst_hbm.at[pl.ds(base, CHUNK)], dbufs[b],
                             dsems[b])
        plsc.subcore_barrier()

        def body(j, carry):
            for b in range(NBUF):
                i = j * NBUF + b
                pltpu.make_async_copy(dst_hbm.at[pl.ds(0, CHUNK)], dbufs[b],
                                      dsems[b]).wait()
                pltpu.sync_copy(ones_v, acc.at[dbufs[b]], add=True)
                nxt = i + NBUF

                @pl.when(nxt < FULL_CHUNKS)
                def _():
                    base = pl.multiple_of(ebase + nxt * CHUNK, 8)
                    pltpu.async_copy(dst_hbm.at[pl.ds(base, CHUNK)],
                                     dbufs[b], dsems[b])

            return carry

        lax.fori_loop(0, FULL_CHUNKS // NBUF, body, 0)
        rbase = pl.multiple_of(ebase + FULL_CHUNKS * CHUNK, 8)
        pltpu.sync_copy(dst_hbm.at[pl.ds(rbase, REM)], idx_r)
        pltpu.sync_copy(ones_r, acc.at[idx_r], add=True)
        plsc.subcore_barrier()
        pltpu.sync_copy(acc.at[pl.ds(row0, ROWS_PER_TILE)],
                        out_hbm.at[cid, pl.ds(row0, ROWS_PER_TILE)])

    return deg


@functools.lru_cache(maxsize=None)
def _prop_kernel(d: int):
    @functools.partial(
        pl.kernel,
        mesh=_mesh(),
        out_type=jax.ShapeDtypeStruct((NC, N_PAD, d), jnp.float32),
        scratch_types=(
            [pltpu.VMEM((EPW,), jnp.int32)]
            + [pltpu.VMEM((CHUNK,), jnp.int32) for _ in range(NBUF)]
            + [pltpu.VMEM((CHUNK, d), jnp.float32) for _ in range(NBUF)]
            + [pltpu.VMEM((REM,), jnp.int32),
               pltpu.VMEM((REM,), jnp.int32),
               pltpu.VMEM((REM, d), jnp.float32),
               pltpu.VMEM_SHARED((N_PAD, d), jnp.float32)]
            + [pltpu.SemaphoreType.DMA for _ in range(2 * NBUF + 1)]
        ),
        compiler_params=pltpu.CompilerParams(use_tc_tiling_on_sc=False),
    )
    def prop(hs_hbm, src_hbm, dst_hbm, zeros_hbm, out_hbm,
             sidx_all, d0, d1, d2, r0, r1, r2,
             sidx_r, didx_r, rows_r, acc,
             sd0, sd1, sd2, sg0, sg1, sg2, sem):
        dbufs = (d0, d1, d2)
        rbufs = (r0, r1, r2)
        dsems = (sd0, sd1, sd2)
        gsems = (sg0, sg1, sg2)
        cid = lax.axis_index("c")
        sid = lax.axis_index("s")
        wid = sid * NC + cid
        row0 = sid * ROWS_PER_TILE
        pltpu.sync_copy(zeros_hbm.at[pl.ds(row0, ROWS_PER_TILE)],
                        acc.at[pl.ds(row0, ROWS_PER_TILE)])
        ebase = wid * EPW
        pltpu.sync_copy(src_hbm.at[pl.ds(ebase, EPW)], sidx_all)
        for b in range(NBUF):
            base = pl.multiple_of(ebase + b * CHUNK, 8)
            pltpu.async_copy(dst_hbm.at[pl.ds(base, CHUNK)], dbufs[b],
                             dsems[b])
            pltpu.async_copy(hs_hbm.at[sidx_all.at[pl.ds(b * CHUNK, CHUNK)]],
                             rbufs[b], gsems[b])
        plsc.subcore_barrier()

        def body(j, carry):
            for b in range(NBUF):
                i = j * NBUF + b
                pltpu.make_async_copy(dst_hbm.at[pl.ds(0, CHUNK)], dbufs[b],
                                      dsems[b]).wait()
                pltpu.make_async_copy(
                    hs_hbm.at[sidx_all.at[pl.ds(0, CHUNK)]], rbufs[b],
                    gsems[b]).wait()
                pltpu.sync_copy(rbufs[b], acc.at[dbufs[b]], add=True)
                nxt = i + NBUF

                @pl.when(nxt < FULL_CHUNKS)
                def _():
                    base = pl.multiple_of(ebase + nxt * CHUNK, 8)
                    pltpu.async_copy(dst_hbm.at[pl.ds(base, CHUNK)],
                                     dbufs[b], dsems[b])
                    pltpu.async_copy(
                        hs_hbm.at[sidx_all.at[pl.ds(nxt * CHUNK, CHUNK)]],
                        rbufs[b], gsems[b])

            return carry

        lax.fori_loop(0, FULL_CHUNKS // NBUF, body, 0)
        rbase = pl.multiple_of(ebase + FULL_CHUNKS * CHUNK, 8)
        pltpu.sync_copy(src_hbm.at[pl.ds(rbase, REM)], sidx_r)
        pltpu.sync_copy(dst_hbm.at[pl.ds(rbase, REM)], didx_r)
        pltpu.async_copy(hs_hbm.at[sidx_r], rows_r, sem).wait()
        pltpu.sync_copy(rows_r, acc.at[didx_r], add=True)
        plsc.subcore_barrier()
        pltpu.sync_copy(acc.at[pl.ds(row0, ROWS_PER_TILE)],
                        out_hbm.at[cid, pl.ds(row0, ROWS_PER_TILE)])

    return prop


def _mm_scale(x_ref, w_ref, h0_ref, h1_ref, hs_ref, dis_ref):
    dis = lax.rsqrt(h0_ref[...] + h1_ref[...] + 1.0)
    dis_ref[...] = dis
    hs_ref[...] = jnp.dot(x_ref[...], w_ref[...],
                          preferred_element_type=jnp.float32) * dis


def _layer2(p_ref, hs_ref, dis_ref, b1_ref, w2_ref, out_ref):
    agg = (p_ref[0] + p_ref[1] + hs_ref[...]) * dis_ref[...] + b1_ref[...]
    h = jnp.maximum(agg, 0.0)
    out_ref[...] = jnp.dot(h, w2_ref[...],
                           preferred_element_type=jnp.float32) * dis_ref[...]


def _final(q_ref, hs_ref, dis_ref, b2_ref, out_ref):
    out_ref[...] = ((q_ref[0] + q_ref[1] + hs_ref[...]) * dis_ref[...]
                    + b2_ref[...])


def kernel(x, edge_index, W1, b1, W2, b2):
    ei = edge_index.astype(jnp.int32)
    src = ei[0]
    dst = ei[1]
    x_p = jnp.pad(x, ((0, N_PAD - N_NODES), (0, 0)))
    z1 = jnp.zeros((N_PAD,), jnp.float32)
    zh = jnp.zeros((N_PAD, D_HID), jnp.float32)
    zo = jnp.zeros((N_PAD, D_OUT), jnp.float32)

    hist = _deg_kernel()(dst, z1)
    h0 = hist[0].reshape(N_PAD, 1)
    h1 = hist[1].reshape(N_PAD, 1)

    hs1, dis = pl.pallas_call(
        _mm_scale,
        out_shape=[jax.ShapeDtypeStruct((N_PAD, D_HID), jnp.float32),
                   jax.ShapeDtypeStruct((N_PAD, 1), jnp.float32)],
    )(x_p, W1, h0, h1)

    p = _prop_kernel(D_HID)(hs1, src, dst, zh)

    hs2 = pl.pallas_call(
        _layer2,
        out_shape=jax.ShapeDtypeStruct((N_PAD, D_OUT), jnp.float32),
    )(p, hs1, dis, b1.reshape(1, D_HID), W2)

    q = _prop_kernel(D_OUT)(hs2, src, dst, zo)

    out = pl.pallas_call(
        _final,
        out_shape=jax.ShapeDtypeStruct((N_PAD, D_OUT), jnp.float32),
    )(q, hs2, dis, b2.reshape(1, D_OUT))
    return out[:N_NODES]


# speedup vs baseline: 48.5358x; 1.0821x over previous
import functools

import jax
import jax.numpy as jnp
from jax import lax
from jax.experimental import pallas as pl
from jax.experimental.pallas import tpu as pltpu
from jax.experimental.pallas import tpu_sc as plsc

N_NODES = 10000
N_PAD = 10240
N_EDGES = 320000
D_IN = 128
D_HID = 64
D_OUT = 32

NC = 2
NS = 16
NW = NC * NS
EPW = N_EDGES // NW
CHUNK = 128
FULL_CHUNKS = EPW // CHUNK
REM = EPW - FULL_CHUNKS * CHUNK
ROWS_PER_TILE = N_PAD // NS
NBUF = 6


def _mesh():
    return plsc.VectorSubcoreMesh(core_axis_name="c", subcore_axis_name="s")


@functools.lru_cache(maxsize=None)
def _deg_kernel():
    @functools.partial(
        pl.kernel,
        mesh=_mesh(),
        out_type=jax.ShapeDtypeStruct((NC, N_PAD), jnp.float32),
        scratch_types=(
            [pltpu.VMEM((CHUNK,), jnp.float32)]
            + [pltpu.VMEM((CHUNK,), jnp.int32) for _ in range(NBUF)]
            + [pltpu.VMEM((REM,), jnp.float32),
               pltpu.VMEM((REM,), jnp.int32),
               pltpu.VMEM_SHARED((N_PAD,), jnp.float32)]
            + [pltpu.SemaphoreType.DMA for _ in range(NBUF)]
        ),
    )
    def deg(dst_hbm, zeros_hbm, out_hbm, ones_v, d0, d1, d2, d3, d4, d5,
            ones_r, idx_r, acc, sd0, sd1, sd2, sd3, sd4, sd5):
        dbufs = (d0, d1, d2, d3, d4, d5)
        dsems = (sd0, sd1, sd2, sd3, sd4, sd5)
        cid = lax.axis_index("c")
        sid = lax.axis_index("s")
        wid = sid * NC + cid
        for j in range(CHUNK // 16):
            ones_v[pl.ds(j * 16, 16)] = jnp.full((16,), 1.0, jnp.float32)
        ones_r[...] = jnp.full((REM,), 1.0, jnp.float32)
        row0 = sid * ROWS_PER_TILE
        pltpu.sync_copy(zeros_hbm.at[pl.ds(row0, ROWS_PER_TILE)],
                        acc.at[pl.ds(row0, ROWS_PER_TILE)])
        ebase = wid * EPW
        for b in range(NBUF):
            base = pl.multiple_of(ebase + b * CHUNK, 8)
            pltpu.async_copy(dst_hbm.at[pl.ds(base, CHUNK)], dbufs[b],
                             dsems[b])
        plsc.subcore_barrier()

        def body(j, carry):
            for b in range(NBUF):
                i = j * NBUF + b
                pltpu.make_async_copy(dst_hbm.at[pl.ds(0, CHUNK)], dbufs[b],
                                      dsems[b]).wait()
                pltpu.sync_copy(ones_v, acc.at[dbufs[b]], add=True)
                nxt = i + NBUF

                @pl.when(nxt < FULL_CHUNKS)
                def _():
                    base = pl.multiple_of(ebase + nxt * CHUNK, 8)
                    pltpu.async_copy(dst_hbm.at[pl.ds(base, CHUNK)],
                                     dbufs[b], dsems[b])

            return carry

        lax.fori_loop(0, FULL_CHUNKS // NBUF, body, 0)
        rbase = pl.multiple_of(ebase + FULL_CHUNKS * CHUNK, 8)
        pltpu.sync_copy(dst_hbm.at[pl.ds(rbase, REM)], idx_r)
        pltpu.sync_copy(ones_r, acc.at[idx_r], add=True)
        plsc.subcore_barrier()
        pltpu.sync_copy(acc.at[pl.ds(row0, ROWS_PER_TILE)],
                        out_hbm.at[cid, pl.ds(row0, ROWS_PER_TILE)])

    return deg


@functools.lru_cache(maxsize=None)
def _prop_kernel(d: int):
    @functools.partial(
        pl.kernel,
        mesh=_mesh(),
        out_type=jax.ShapeDtypeStruct((NC, N_PAD, d), jnp.float32),
        scratch_types=(
            [pltpu.VMEM((EPW,), jnp.int32)]
            + [pltpu.VMEM((CHUNK,), jnp.int32) for _ in range(NBUF)]
            + [pltpu.VMEM((CHUNK, d), jnp.float32) for _ in range(NBUF)]
            + [pltpu.VMEM((REM,), jnp.int32),
               pltpu.VMEM((REM,), jnp.int32),
               pltpu.VMEM((REM, d), jnp.float32),
               pltpu.VMEM_SHARED((N_PAD, d), jnp.float32)]
            + [pltpu.SemaphoreType.DMA for _ in range(2 * NBUF + 1)]
        ),
        compiler_params=pltpu.CompilerParams(use_tc_tiling_on_sc=False),
    )
    def prop(hs_hbm, src_hbm, dst_hbm, zeros_hbm, out_hbm,
             sidx_all, d0, d1, d2, d3, d4, d5, r0, r1, r2, r3, r4, r5,
             sidx_r, didx_r, rows_r, acc,
             sd0, sd1, sd2, sd3, sd4, sd5, sg0, sg1, sg2, sg3, sg4, sg5, sem):
        dbufs = (d0, d1, d2, d3, d4, d5)
        rbufs = (r0, r1, r2, r3, r4, r5)
        dsems = (sd0, sd1, sd2, sd3, sd4, sd5)
        gsems = (sg0, sg1, sg2, sg3, sg4, sg5)
        cid = lax.axis_index("c")
        sid = lax.axis_index("s")
        wid = sid * NC + cid
        row0 = sid * ROWS_PER_TILE
        pltpu.sync_copy(zeros_hbm.at[pl.ds(row0, ROWS_PER_TILE)],
                        acc.at[pl.ds(row0, ROWS_PER_TILE)])
        ebase = wid * EPW
        pltpu.sync_copy(src_hbm.at[pl.ds(ebase, EPW)], sidx_all)
        for b in range(NBUF):
            base = pl.multiple_of(ebase + b * CHUNK, 8)
            pltpu.async_copy(dst_hbm.at[pl.ds(base, CHUNK)], dbufs[b],
                             dsems[b])
            pltpu.async_copy(hs_hbm.at[sidx_all.at[pl.ds(b * CHUNK, CHUNK)]],
                             rbufs[b], gsems[b])
        plsc.subcore_barrier()

        def body(j, carry):
            for b in range(NBUF):
                i = j * NBUF + b
                pltpu.make_async_copy(dst_hbm.at[pl.ds(0, CHUNK)], dbufs[b],
                                      dsems[b]).wait()
                pltpu.make_async_copy(
                    hs_hbm.at[sidx_all.at[pl.ds(0, CHUNK)]], rbufs[b],
                    gsems[b]).wait()
                pltpu.sync_copy(rbufs[b], acc.at[dbufs[b]], add=True)
                nxt = i + NBUF

                @pl.when(nxt < FULL_CHUNKS)
                def _():
                    base = pl.multiple_of(ebase + nxt * CHUNK, 8)
                    pltpu.async_copy(dst_hbm.at[pl.ds(base, CHUNK)],
                                     dbufs[b], dsems[b])
                    pltpu.async_copy(
                        hs_hbm.at[sidx_all.at[pl.ds(nxt * CHUNK, CHUNK)]],
                        rbufs[b], gsems[b])

            return carry

        lax.fori_loop(0, FULL_CHUNKS // NBUF, body, 0)
        rbase = pl.multiple_of(ebase + FULL_CHUNKS * CHUNK, 8)
        pltpu.sync_copy(src_hbm.at[pl.ds(rbase, REM)], sidx_r)
        pltpu.sync_copy(dst_hbm.at[pl.ds(rbase, REM)], didx_r)
        pltpu.async_copy(hs_hbm.at[sidx_r], rows_r, sem).wait()
        pltpu.sync_copy(rows_r, acc.at[didx_r], add=True)
        plsc.subcore_barrier()
        pltpu.sync_copy(acc.at[pl.ds(row0, ROWS_PER_TILE)],
                        out_hbm.at[cid, pl.ds(row0, ROWS_PER_TILE)])

    return prop


def _mm_scale(x_ref, w_ref, h0_ref, h1_ref, hs_ref, dis_ref):
    dis = lax.rsqrt(h0_ref[...] + h1_ref[...] + 1.0)
    dis_ref[...] = dis
    hs_ref[...] = jnp.dot(x_ref[...], w_ref[...],
                          preferred_element_type=jnp.float32) * dis


def _layer2(p_ref, hs_ref, dis_ref, b1_ref, w2_ref, out_ref):
    agg = (p_ref[0] + p_ref[1] + hs_ref[...]) * dis_ref[...] + b1_ref[...]
    h = jnp.maximum(agg, 0.0)
    out_ref[...] = jnp.dot(h, w2_ref[...],
                           preferred_element_type=jnp.float32) * dis_ref[...]


def _final(q_ref, hs_ref, dis_ref, b2_ref, out_ref):
    out_ref[...] = ((q_ref[0] + q_ref[1] + hs_ref[...]) * dis_ref[...]
                    + b2_ref[...])


def kernel(x, edge_index, W1, b1, W2, b2):
    ei = edge_index.astype(jnp.int32)
    src = ei[0]
    dst = ei[1]
    x_p = jnp.pad(x, ((0, N_PAD - N_NODES), (0, 0)))
    z1 = jnp.zeros((N_PAD,), jnp.float32)
    zh = jnp.zeros((N_PAD, D_HID), jnp.float32)
    zo = jnp.zeros((N_PAD, D_OUT), jnp.float32)

    hist = _deg_kernel()(dst, z1)
    h0 = hist[0].reshape(N_PAD, 1)
    h1 = hist[1].reshape(N_PAD, 1)

    hs1, dis = pl.pallas_call(
        _mm_scale,
        out_shape=[jax.ShapeDtypeStruct((N_PAD, D_HID), jnp.float32),
                   jax.ShapeDtypeStruct((N_PAD, 1), jnp.float32)],
    )(x_p, W1, h0, h1)

    p = _prop_kernel(D_HID)(hs1, src, dst, zh)

    hs2 = pl.pallas_call(
        _layer2,
        out_shape=jax.ShapeDtypeStruct((N_PAD, D_OUT), jnp.float32),
    )(p, hs1, dis, b1.reshape(1, D_HID), W2)

    q = _prop_kernel(D_OUT)(hs2, src, dst, zo)

    out = pl.pallas_call(
        _final,
        out_shape=jax.ShapeDtypeStruct((N_PAD, D_OUT), jnp.float32),
    )(q, hs2, dis, b2.reshape(1, D_OUT))
    return out[:N_NODES]


# speedup vs baseline: 48.6966x; 1.0033x over previous
import functools

import jax
import jax.numpy as jnp
from jax import lax
from jax.experimental import pallas as pl
from jax.experimental.pallas import tpu as pltpu
from jax.experimental.pallas import tpu_sc as plsc

N_NODES = 10000
N_PAD = 10240
N_EDGES = 320000
D_IN = 128
D_HID = 64
D_OUT = 32

NC = 2
NS = 16
NW = NC * NS
EPW = N_EDGES // NW
CHUNK = 128
FULL_CHUNKS = EPW // CHUNK
REM = EPW - FULL_CHUNKS * CHUNK
ROWS_PER_TILE = N_PAD // NS
OUT_ROWS_PER_TILE = N_NODES // NS
NBUF = 6


def _mesh():
    return plsc.VectorSubcoreMesh(core_axis_name="c", subcore_axis_name="s")


@functools.lru_cache(maxsize=None)
def _deg_kernel():
    @functools.partial(
        pl.kernel,
        mesh=_mesh(),
        out_type=jax.ShapeDtypeStruct((NC, N_PAD), jnp.float32),
        scratch_types=(
            [pltpu.VMEM((CHUNK,), jnp.float32)]
            + [pltpu.VMEM((CHUNK,), jnp.int32) for _ in range(NBUF)]
            + [pltpu.VMEM((REM,), jnp.float32),
               pltpu.VMEM((REM,), jnp.int32),
               pltpu.VMEM_SHARED((N_PAD,), jnp.float32)]
            + [pltpu.SemaphoreType.DMA for _ in range(NBUF)]
        ),
    )
    def deg(dst_hbm, zeros_hbm, out_hbm, ones_v, d0, d1, d2, d3, d4, d5,
            ones_r, idx_r, acc, sd0, sd1, sd2, sd3, sd4, sd5):
        dbufs = (d0, d1, d2, d3, d4, d5)
        dsems = (sd0, sd1, sd2, sd3, sd4, sd5)
        cid = lax.axis_index("c")
        sid = lax.axis_index("s")
        wid = sid * NC + cid
        for j in range(CHUNK // 16):
            ones_v[pl.ds(j * 16, 16)] = jnp.full((16,), 1.0, jnp.float32)
        ones_r[...] = jnp.full((REM,), 1.0, jnp.float32)
        row0 = sid * ROWS_PER_TILE
        pltpu.sync_copy(zeros_hbm.at[pl.ds(row0, ROWS_PER_TILE)],
                        acc.at[pl.ds(row0, ROWS_PER_TILE)])
        ebase = wid * EPW
        for b in range(NBUF):
            base = pl.multiple_of(ebase + b * CHUNK, 8)
            pltpu.async_copy(dst_hbm.at[pl.ds(base, CHUNK)], dbufs[b],
                             dsems[b])
        plsc.subcore_barrier()

        def body(j, carry):
            for b in range(NBUF):
                i = j * NBUF + b
                pltpu.make_async_copy(dst_hbm.at[pl.ds(0, CHUNK)], dbufs[b],
                                      dsems[b]).wait()
                pltpu.sync_copy(ones_v, acc.at[dbufs[b]], add=True)
                nxt = i + NBUF

                @pl.when(nxt < FULL_CHUNKS)
                def _():
                    base = pl.multiple_of(ebase + nxt * CHUNK, 8)
                    pltpu.async_copy(dst_hbm.at[pl.ds(base, CHUNK)],
                                     dbufs[b], dsems[b])

            return carry

        lax.fori_loop(0, FULL_CHUNKS // NBUF, body, 0)
        rbase = pl.multiple_of(ebase + FULL_CHUNKS * CHUNK, 8)
        pltpu.sync_copy(dst_hbm.at[pl.ds(rbase, REM)], idx_r)
        pltpu.sync_copy(ones_r, acc.at[idx_r], add=True)
        plsc.subcore_barrier()
        pltpu.sync_copy(acc.at[pl.ds(row0, ROWS_PER_TILE)],
                        out_hbm.at[cid, pl.ds(row0, ROWS_PER_TILE)])

    return deg


@functools.lru_cache(maxsize=None)
def _prop_kernel(d: int):
    @functools.partial(
        pl.kernel,
        mesh=_mesh(),
        out_type=jax.ShapeDtypeStruct((NC, N_NODES, d), jnp.float32),
        scratch_types=(
            [pltpu.VMEM((EPW,), jnp.int32)]
            + [pltpu.VMEM((CHUNK,), jnp.int32) for _ in range(NBUF)]
            + [pltpu.VMEM((CHUNK, d), jnp.float32) for _ in range(NBUF)]
            + [pltpu.VMEM((REM,), jnp.int32),
               pltpu.VMEM((REM,), jnp.int32),
               pltpu.VMEM((REM, d), jnp.float32),
               pltpu.VMEM_SHARED((N_NODES, d), jnp.float32)]
            + [pltpu.SemaphoreType.DMA for _ in range(2 * NBUF + 1)]
        ),
        compiler_params=pltpu.CompilerParams(use_tc_tiling_on_sc=False),
    )
    def prop(hs_hbm, src_hbm, dst_hbm, zeros_hbm, out_hbm,
             sidx_all, d0, d1, d2, d3, d4, d5, r0, r1, r2, r3, r4, r5,
             sidx_r, didx_r, rows_r, acc,
             sd0, sd1, sd2, sd3, sd4, sd5, sg0, sg1, sg2, sg3, sg4, sg5, sem):
        dbufs = (d0, d1, d2, d3, d4, d5)
        rbufs = (r0, r1, r2, r3, r4, r5)
        dsems = (sd0, sd1, sd2, sd3, sd4, sd5)
        gsems = (sg0, sg1, sg2, sg3, sg4, sg5)
        cid = lax.axis_index("c")
        sid = lax.axis_index("s")
        wid = sid * NC + cid
        row0 = sid * OUT_ROWS_PER_TILE
        pltpu.sync_copy(zeros_hbm.at[pl.ds(row0, OUT_ROWS_PER_TILE)],
                        acc.at[pl.ds(row0, OUT_ROWS_PER_TILE)])
        ebase = wid * EPW
        pltpu.sync_copy(src_hbm.at[pl.ds(ebase, EPW)], sidx_all)
        for b in range(NBUF):
            base = pl.multiple_of(ebase + b * CHUNK, 8)
            pltpu.async_copy(dst_hbm.at[pl.ds(base, CHUNK)], dbufs[b],
                             dsems[b])
            pltpu.async_copy(hs_hbm.at[sidx_all.at[pl.ds(b * CHUNK, CHUNK)]],
                             rbufs[b], gsems[b])
        plsc.subcore_barrier()

        def body(j, carry):
            for b in range(NBUF):
                i = j * NBUF + b
                pltpu.make_async_copy(dst_hbm.at[pl.ds(0, CHUNK)], dbufs[b],
                                      dsems[b]).wait()
                pltpu.make_async_copy(
                    hs_hbm.at[sidx_all.at[pl.ds(0, CHUNK)]], rbufs[b],
                    gsems[b]).wait()
                pltpu.sync_copy(rbufs[b], acc.at[dbufs[b]], add=True)
                nxt = i + NBUF

                @pl.when(nxt < FULL_CHUNKS)
                def _():
                    base = pl.multiple_of(ebase + nxt * CHUNK, 8)
                    pltpu.async_copy(dst_hbm.at[pl.ds(base, CHUNK)],
                                     dbufs[b], dsems[b])
                    pltpu.async_copy(
                        hs_hbm.at[sidx_all.at[pl.ds(nxt * CHUNK, CHUNK)]],
                        rbufs[b], gsems[b])

            return carry

        lax.fori_loop(0, FULL_CHUNKS // NBUF, body, 0)
        rbase = pl.multiple_of(ebase + FULL_CHUNKS * CHUNK, 8)
        pltpu.sync_copy(src_hbm.at[pl.ds(rbase, REM)], sidx_r)
        pltpu.sync_copy(dst_hbm.at[pl.ds(rbase, REM)], didx_r)
        pltpu.async_copy(hs_hbm.at[sidx_r], rows_r, sem).wait()
        pltpu.sync_copy(rows_r, acc.at[didx_r], add=True)
        plsc.subcore_barrier()
        pltpu.sync_copy(acc.at[pl.ds(row0, OUT_ROWS_PER_TILE)],
                        out_hbm.at[cid, pl.ds(row0, OUT_ROWS_PER_TILE)])

    return prop


def _mm_scale(x_ref, w_ref, h0_ref, h1_ref, hs_ref, dis_ref):
    dis = lax.rsqrt(h0_ref[...] + h1_ref[...] + 1.0)
    dis_ref[...] = dis
    hs_ref[...] = jnp.dot(x_ref[...], w_ref[...],
                          preferred_element_type=jnp.float32) * dis


def _layer2(p_ref, hs_ref, dis_ref, b1_ref, w2_ref, out_ref):
    agg = (p_ref[0] + p_ref[1] + hs_ref[...]) * dis_ref[...] + b1_ref[...]
    h = jnp.maximum(agg, 0.0)
    out_ref[...] = jnp.dot(h, w2_ref[...],
                           preferred_element_type=jnp.float32) * dis_ref[...]


def _final(q_ref, hs_ref, dis_ref, b2_ref, out_ref):
    out_ref[...] = ((q_ref[0] + q_ref[1] + hs_ref[...]) * dis_ref[...]
                    + b2_ref[...])


def kernel(x, edge_index, W1, b1, W2, b2):
    ei = edge_index.astype(jnp.int32)
    src = ei[0]
    dst = ei[1]
    z1 = jnp.zeros((N_PAD,), jnp.float32)
    zh = jnp.zeros((N_NODES, D_HID), jnp.float32)
    zo = jnp.zeros((N_NODES, D_OUT), jnp.float32)

    hist = _deg_kernel()(dst, z1)
    h0 = hist[0, :N_NODES].reshape(N_NODES, 1)
    h1 = hist[1, :N_NODES].reshape(N_NODES, 1)

    hs1, dis = pl.pallas_call(
        _mm_scale,
        out_shape=[jax.ShapeDtypeStruct((N_NODES, D_HID), jnp.float32),
                   jax.ShapeDtypeStruct((N_NODES, 1), jnp.float32)],
    )(x, W1, h0, h1)

    p = _prop_kernel(D_HID)(hs1, src, dst, zh)

    hs2 = pl.pallas_call(
        _layer2,
        out_shape=jax.ShapeDtypeStruct((N_NODES, D_OUT), jnp.float32),
    )(p, hs1, dis, b1.reshape(1, D_HID), W2)

    q = _prop_kernel(D_OUT)(hs2, src, dst, zo)

    out = pl.pallas_call(
        _final,
        out_shape=jax.ShapeDtypeStruct((N_NODES, D_OUT), jnp.float32),
    )(q, hs2, dis, b2.reshape(1, D_OUT))
    return out
